# baseline (speedup 1.0000x reference)
import jax
import jax.numpy as jnp
from jax.experimental import pallas as pl
from jax.experimental.pallas import tpu as pltpu

N_TOKENS = 16384
D_MODEL = 2048
N_EXPERTS = 16
K = 2
BLOCK_T = 2048


def _router_body(x_ref, w_ref, gates_ref, vals_ref, inds_ref):
    x = x_ref[...]
    w = w_ref[...]
    logits_t = jax.lax.dot_general(
        w, x, (((1,), (1,)), ((), ())), preferred_element_type=jnp.float32)

    m = jnp.max(logits_t, axis=0, keepdims=True)
    e = jnp.exp(logits_t - m)
    s = jnp.sum(e, axis=0, keepdims=True)
    gates_t = e / s
    gates_ref[...] = gates_t.T

    iota = jax.lax.broadcasted_iota(jnp.int32, gates_t.shape, 0)
    m1 = jnp.max(gates_t, axis=0, keepdims=True)
    i1 = jnp.min(jnp.where(gates_t == m1, iota, N_EXPERTS), axis=0, keepdims=True)
    g2 = jnp.where(iota == i1, -jnp.inf, gates_t)
    m2 = jnp.max(g2, axis=0, keepdims=True)
    i2 = jnp.min(jnp.where(g2 == m2, iota, N_EXPERTS), axis=0, keepdims=True)

    vals_ref[...] = jnp.concatenate([m1, m2], axis=0).T
    inds_ref[...] = jnp.concatenate([i1, i2], axis=0).T


def kernel(hidden_states, gate_w, noise_w):
    x_in = hidden_states + (noise_w[0, 0] - noise_w[0, 0])

    grid = (N_TOKENS // BLOCK_T,)
    gates, vals, inds = pl.pallas_call(
        _router_body,
        grid=grid,
        in_specs=[
            pl.BlockSpec((BLOCK_T, D_MODEL), lambda i: (i, 0)),
            pl.BlockSpec((N_EXPERTS, D_MODEL), lambda i: (0, 0)),
        ],
        out_specs=[
            pl.BlockSpec((BLOCK_T, N_EXPERTS), lambda i: (i, 0)),
            pl.BlockSpec((BLOCK_T, K), lambda i: (i, 0)),
            pl.BlockSpec((BLOCK_T, K), lambda i: (i, 0)),
        ],
        out_shape=[
            jax.ShapeDtypeStruct((N_TOKENS, N_EXPERTS), jnp.float32),
            jax.ShapeDtypeStruct((N_TOKENS, K), jnp.float32),
            jax.ShapeDtypeStruct((N_TOKENS, K), jnp.int32),
        ],
        compiler_params=pltpu.CompilerParams(
            allow_input_fusion=(True, False),
        ),
    )(x_in, gate_w)
    return vals, inds, gates


# speedup vs baseline: 1.6965x; 1.6965x over previous
import functools

import jax
import jax.numpy as jnp
from jax import lax
from jax.experimental import pallas as pl
from jax.experimental.pallas import tpu as pltpu
from jax.experimental.pallas import tpu_sc as plsc

N_TOKENS = 16384
D_MODEL = 2048
N_EXPERTS = 16
K = 2
BLOCK_T = 1024

NC = 2
NS = 16
NW = NC * NS
TOK_W = 256
GROUPS = TOK_W // 16
NW_TOTAL = N_TOKENS // TOK_W
W_PER_BLOCK = BLOCK_T // TOK_W


def _tc_body(x_ref, w_ref, gates_ref, gatest_ref):
    x = x_ref[...]
    w = w_ref[...]
    logits_t = jax.lax.dot_general(
        w, x, (((1,), (1,)), ((), ())), preferred_element_type=jnp.float32)
    m = jnp.max(logits_t, axis=0, keepdims=True)
    e = jnp.exp(logits_t - m)
    s = jnp.sum(e, axis=0, keepdims=True)
    gates_t = e / s
    gates_ref[...] = gates_t.T
    for wloc in range(W_PER_BLOCK):
        gatest_ref[wloc] = gates_t[:, wloc * TOK_W:(wloc + 1) * TOK_W]


def _tc_matmul_softmax(hidden_states, gate_w):
    grid = (N_TOKENS // BLOCK_T,)
    return pl.pallas_call(
        _tc_body,
        grid=grid,
        in_specs=[
            pl.BlockSpec((BLOCK_T, D_MODEL), lambda i: (i, 0)),
            pl.BlockSpec((N_EXPERTS, D_MODEL), lambda i: (0, 0)),
        ],
        out_specs=[
            pl.BlockSpec((BLOCK_T, N_EXPERTS), lambda i: (i, 0)),
            pl.BlockSpec((W_PER_BLOCK, N_EXPERTS, TOK_W), lambda i: (i, 0, 0)),
        ],
        out_shape=[
            jax.ShapeDtypeStruct((N_TOKENS, N_EXPERTS), jnp.float32),
            jax.ShapeDtypeStruct((NW_TOTAL, N_EXPERTS, TOK_W), jnp.float32),
        ],
    )(hidden_states, gate_w)


@functools.partial(
    pl.kernel,
    mesh=plsc.VectorSubcoreMesh(core_axis_name="c", subcore_axis_name="s"),
    out_type=[
        jax.ShapeDtypeStruct((NW, TOK_W, K), jnp.float32),
        jax.ShapeDtypeStruct((NW, TOK_W, K), jnp.int32),
    ],
    scratch_types=[
        pltpu.VMEM((N_EXPERTS, TOK_W), jnp.float32),
        pltpu.VMEM((TOK_W, K), jnp.float32),
        pltpu.VMEM((TOK_W, K), jnp.int32),
    ],
    compiler_params=pltpu.CompilerParams(needs_layout_passes=False),
)
def _sc_top2(gatest_hbm, vals_hbm, inds_hbm, gbuf, vbuf, ibuf):
    wid = lax.axis_index("s") * NC + lax.axis_index("c")
    pltpu.sync_copy(gatest_hbm.at[wid], gbuf)

    lane = lax.iota(jnp.int32, 16)
    neg_inf = jnp.full((16,), -jnp.inf, jnp.float32)
    for g in range(GROUPS):
        vs = [gbuf[e, pl.ds(g * 16, 16)] for e in range(N_EXPERTS)]
        m1 = vs[0]
        for e in range(1, N_EXPERTS):
            m1 = jnp.maximum(m1, vs[e])
        i1 = jnp.zeros((16,), jnp.int32)
        for e in range(N_EXPERTS - 1, -1, -1):
            i1 = jnp.where(vs[e] == m1, jnp.int32(e), i1)
        m2 = neg_inf
        for e in range(N_EXPERTS):
            m2 = jnp.maximum(m2, jnp.where(i1 == e, neg_inf, vs[e]))
        i2 = jnp.zeros((16,), jnp.int32)
        for e in range(N_EXPERTS - 1, -1, -1):
            i2 = jnp.where((vs[e] == m2) & (i1 != e), jnp.int32(e), i2)

        row = lane + g * 16
        zero = jnp.zeros((16,), jnp.int32)
        one = jnp.ones((16,), jnp.int32)
        plsc.store_scatter(vbuf, [row, zero], m1)
        plsc.store_scatter(vbuf, [row, one], m2)
        plsc.store_scatter(ibuf, [row, zero], i1)
        plsc.store_scatter(ibuf, [row, one], i2)

    pltpu.sync_copy(vbuf, vals_hbm.at[wid])
    pltpu.sync_copy(ibuf, inds_hbm.at[wid])


def kernel(hidden_states, gate_w, noise_w):
    del noise_w
    gates, gates_t4 = _tc_matmul_softmax(hidden_states, gate_w)
    vals_a, inds_a = _sc_top2(gates_t4[:NW])
    vals_b, inds_b = _sc_top2(gates_t4[NW:])
    vals = jnp.concatenate([vals_a, vals_b], axis=0).reshape(N_TOKENS, K)
    inds = jnp.concatenate([inds_a, inds_b], axis=0).reshape(N_TOKENS, K)
    return (vals, inds, gates)


# speedup vs baseline: 2.1957x; 1.2943x over previous
import jax
import jax.numpy as jnp
from jax.experimental import pallas as pl
from jax.experimental.pallas import tpu as pltpu

N_TOKENS = 16384
D_MODEL = 2048
N_EXPERTS = 16
K = 2
BLOCK_T = 1024


def _router_body(x_ref, w_ref, gates_ref, vals_ref, inds_ref):
    x = x_ref[...]
    w = w_ref[...]
    logits_t = jax.lax.dot_general(
        w, x, (((1,), (1,)), ((), ())), preferred_element_type=jnp.float32)

    m = jnp.max(logits_t, axis=0, keepdims=True)
    e = jnp.exp(logits_t - m)
    s = jnp.sum(e, axis=0, keepdims=True)
    gates_t = e / s
    gates_ref[...] = gates_t.T

    iota = jax.lax.broadcasted_iota(jnp.int32, gates_t.shape, 0)
    m1 = jnp.max(gates_t, axis=0, keepdims=True)
    i1 = jnp.min(jnp.where(gates_t == m1, iota, N_EXPERTS), axis=0, keepdims=True)
    g2 = jnp.where(iota == i1, -jnp.inf, gates_t)
    m2 = jnp.max(g2, axis=0, keepdims=True)
    i2 = jnp.min(jnp.where(g2 == m2, iota, N_EXPERTS), axis=0, keepdims=True)

    vals_ref[...] = jnp.concatenate([m1, m2], axis=0).T
    inds_ref[...] = jnp.concatenate([i1, i2], axis=0).T


def kernel(hidden_states, gate_w, noise_w):
    del noise_w

    grid = (N_TOKENS // BLOCK_T,)
    gates, vals, inds = pl.pallas_call(
        _router_body,
        grid=grid,
        in_specs=[
            pl.BlockSpec((BLOCK_T, D_MODEL), lambda i: (i, 0)),
            pl.BlockSpec((N_EXPERTS, D_MODEL), lambda i: (0, 0)),
        ],
        out_specs=[
            pl.BlockSpec((BLOCK_T, N_EXPERTS), lambda i: (i, 0)),
            pl.BlockSpec((BLOCK_T, K), lambda i: (i, 0)),
            pl.BlockSpec((BLOCK_T, K), lambda i: (i, 0)),
        ],
        out_shape=[
            jax.ShapeDtypeStruct((N_TOKENS, N_EXPERTS), jnp.float32),
            jax.ShapeDtypeStruct((N_TOKENS, K), jnp.float32),
            jax.ShapeDtypeStruct((N_TOKENS, K), jnp.int32),
        ],
    )(hidden_states, gate_w)
    return vals, inds, gates
